# baseline (speedup 1.0000x reference)
import functools

import jax
import jax.numpy as jnp
from jax import lax
from jax.experimental import pallas as pl
from jax.experimental.pallas import tpu as pltpu
from jax.experimental.pallas import tpu_sc as plsc

BATCH = 16384
EMBED = 64
NFEAT = 26

_NW = 32
_BPW = BATCH // _NW
_CHUNK = 128
_NCH = _BPW // _CHUNK


def _sc_gather_body(uemb, iemb, uidx, iidx, out_u, out_i,
                    uidx_v, iidx_v, urows, irows, usem, isem):
    wid = lax.axis_index("s") * 2 + lax.axis_index("c")
    base = wid * _BPW
    pltpu.sync_copy(uidx.at[pl.ds(wid * _NCH, _NCH)], uidx_v)
    pltpu.sync_copy(iidx.at[pl.ds(wid * _NCH, _NCH)], iidx_v)
    ops = []
    for j in range(_NCH):
        ops.append(pltpu.async_copy(
            uemb.at[uidx_v.at[j]], urows.at[pl.ds(j * _CHUNK, _CHUNK)], usem))
        ops.append(pltpu.async_copy(
            iemb.at[iidx_v.at[j]], irows.at[pl.ds(j * _CHUNK, _CHUNK)], isem))
    for o in ops:
        o.wait()
    pltpu.sync_copy(urows, out_u.at[pl.ds(base, _BPW)])
    pltpu.sync_copy(irows, out_i.at[pl.ds(base, _BPW)])


@functools.lru_cache(maxsize=1)
def _make_sc_gather():
    return pl.kernel(
        _sc_gather_body,
        mesh=plsc.VectorSubcoreMesh(core_axis_name="c", subcore_axis_name="s"),
        out_type=[
            jax.ShapeDtypeStruct((BATCH, EMBED), jnp.float32),
            jax.ShapeDtypeStruct((BATCH, EMBED), jnp.float32),
        ],
        scratch_types=[
            pltpu.VMEM((_NCH, _CHUNK), jnp.int32),
            pltpu.VMEM((_NCH, _CHUNK), jnp.int32),
            pltpu.VMEM((_BPW, EMBED), jnp.float32),
            pltpu.VMEM((_BPW, EMBED), jnp.float32),
            pltpu.SemaphoreType.DMA,
            pltpu.SemaphoreType.DMA,
        ],
        compiler_params=pltpu.CompilerParams(use_tc_tiling_on_sc=False),
    )


_TC_BLOCK = 2048
_TC_GRID = BATCH // _TC_BLOCK


def _tc_combine_body(ug_ref, ig_ref, uf_ref, if_ref,
                     wu_ref, bu_ref, wi_ref, bi_ref, out_ref):
    pu = jnp.dot(uf_ref[...], wu_ref[...],
                 preferred_element_type=jnp.float32) + bu_ref[...]
    pi = jnp.dot(if_ref[...], wi_ref[...],
                 preferred_element_type=jnp.float32) + bi_ref[...]
    u = ug_ref[...] + pu
    v = ig_ref[...] + pi
    out_ref[...] = jnp.sum(u * v, axis=1, keepdims=True)


_tc_combine = pl.pallas_call(
    _tc_combine_body,
    grid=(_TC_GRID,),
    in_specs=[
        pl.BlockSpec((_TC_BLOCK, EMBED), lambda i: (i, 0)),
        pl.BlockSpec((_TC_BLOCK, EMBED), lambda i: (i, 0)),
        pl.BlockSpec((_TC_BLOCK, NFEAT), lambda i: (i, 0)),
        pl.BlockSpec((_TC_BLOCK, NFEAT), lambda i: (i, 0)),
        pl.BlockSpec((NFEAT, EMBED), lambda i: (0, 0)),
        pl.BlockSpec((1, EMBED), lambda i: (0, 0)),
        pl.BlockSpec((NFEAT, EMBED), lambda i: (0, 0)),
        pl.BlockSpec((1, EMBED), lambda i: (0, 0)),
    ],
    out_specs=pl.BlockSpec((_TC_BLOCK, 1), lambda i: (i, 0)),
    out_shape=jax.ShapeDtypeStruct((BATCH, 1), jnp.float32),
)


def kernel(user_indices, item_indices, user_features, item_features,
           user_emb, item_emb, Wu, bu, Wi, bi):
    uidx = user_indices.astype(jnp.int32).reshape(BATCH // _CHUNK, _CHUNK)
    iidx = item_indices.astype(jnp.int32).reshape(BATCH // _CHUNK, _CHUNK)
    ug, ig = _make_sc_gather()(user_emb, item_emb, uidx, iidx)
    out = _tc_combine(ug, ig, user_features, item_features,
                      Wu, bu.reshape(1, EMBED), Wi, bi.reshape(1, EMBED))
    return out.reshape(BATCH)
